# baseline (speedup 1.0000x reference)
import functools

import jax
import jax.numpy as jnp
from jax import lax
from jax.experimental import pallas as pl
from jax.experimental.pallas import tpu as pltpu
from jax.experimental.pallas import tpu_sc as plsc

GAMMA = 12.0
HIDDEN = 64
BATCH = 16384
LANES = 16
NUM_WORKERS = 32
B_PER_W = BATCH // NUM_WORKERS
NENTITY = 1000000
NTILES = (NENTITY + 127) // 128
BIG_W = NTILES - (NTILES // NUM_WORKERS) * NUM_WORKERS
TILES_SMALL = NTILES // NUM_WORKERS
CHUNK_COLS = 512
N_ENTRIES = 2 * BATCH
STAGE_ROWS = N_ENTRIES + LANES
IDX_CHUNK = 4096


def _worker_range(wid):
  tiles = jnp.where(wid < BIG_W, TILES_SMALL + 1, TILES_SMALL)
  start = jnp.where(
      wid < BIG_W,
      wid * (TILES_SMALL + 1),
      BIG_W * (TILES_SMALL + 1) + (wid - BIG_W) * TILES_SMALL)
  return start * 128, tiles * 128


def _make_gather_kernel():
  mesh = plsc.VectorSubcoreMesh(core_axis_name="c", subcore_axis_name="s")

  @functools.partial(
      pl.kernel,
      mesh=mesh,
      compiler_params=pltpu.CompilerParams(needs_layout_passes=False),
      out_type=jax.ShapeDtypeStruct((STAGE_ROWS, 128), jnp.float32),
      scratch_types=[
          pltpu.VMEM((IDX_CHUNK,), jnp.int32),
          pltpu.VMEM((N_ENTRIES + 16,), jnp.int32),
          pltpu.VMEM((HIDDEN, CHUNK_COLS), jnp.float32),
          pltpu.VMEM((HIDDEN, CHUNK_COLS), jnp.float32),
          pltpu.VMEM((LANES, 128), jnp.float32),
          pltpu.VMEM((LANES, 128), jnp.float32),
          pltpu.VMEM((32 * 128,), jnp.float32),
          pltpu.VMEM((LANES,), jnp.int32),
          pltpu.SemaphoreType.DMA,
          pltpu.SemaphoreType.DMA,
          pltpu.SemaphoreType.DMA,
      ],
  )
  def gather_kernel(heads_hbm, tails_hbm, ent_t_hbm, tail_hbm, stage_hbm,
                    idxbuf, wl, cb0, cb1, asm, asm1, tailbuf, ctmp,
                    sem0, sem1, sem2):
    wid = lax.axis_index("s") * 2 + lax.axis_index("c")
    lo, rlen = _worker_range(wid)
    lane = lax.iota(jnp.int32, LANES)
    dump_ids = jnp.full((LANES,), N_ENTRIES, jnp.int32) + lane
    pltpu.sync_copy(tail_hbm, tailbuf)

    def scan_table(tab_hbm, entry_base, cnt0):
      def piece(pc, cntv):
        pltpu.sync_copy(tab_hbm.at[pl.ds(pc * IDX_CHUNK, IDX_CHUNK)], idxbuf)
        def vec(v, cntv):
          idx16 = idxbuf[pl.ds(v * LANES, LANES)]
          off = idx16 - lo
          m = (off >= 0) & (off < rlen)
          packed = lax.shift_left(off, 16) | (
              entry_base + pc * IDX_CHUNK + v * LANES + lane)
          pos = cntv + plsc.cumsum(m.astype(jnp.int32)) - 1
          plsc.store_scatter(wl, [pos], packed, mask=m)
          return cntv + plsc.all_reduce_population_count(m)
        return lax.fori_loop(0, IDX_CHUNK // LANES, vec, cntv)
      return lax.fori_loop(0, BATCH // IDX_CHUNK, piece, cnt0)

    cntv = scan_table(heads_hbm, 0, jnp.zeros((LANES,), jnp.int32))
    cntv = scan_table(tails_hbm, BATCH, cntv)
    cnt = jnp.sum(jnp.where(lane == 0, cntv, 0))
    nvec = (cnt + LANES - 1) // LANES

    nch = (rlen + CHUNK_COLS - 1) // CHUNK_COLS
    PART_COLS = 384
    TAIL_START = NENTITY - 64

    def fire(j, cb, sem):
      col0 = lo + j * CHUNK_COLS

      @pl.when(col0 + CHUNK_COLS <= NENTITY)
      def _full():
        pltpu.async_copy(ent_t_hbm.at[:, pl.ds(col0, CHUNK_COLS)], cb, sem)

      @pl.when(col0 + CHUNK_COLS > NENTITY)
      def _part():
        pltpu.async_copy(ent_t_hbm.at[:, pl.ds(col0, PART_COLS)],
                         cb.at[:, pl.ds(0, PART_COLS)], sem)

    def process(j, cb, sem, carry):
      col0 = lo + j * CHUNK_COLS

      @pl.when(col0 + CHUNK_COLS <= NENTITY)
      def _wfull():
        pltpu.make_async_copy(
            ent_t_hbm.at[:, pl.ds(0, CHUNK_COLS)], cb, sem).wait()

      @pl.when(col0 + CHUNK_COLS > NENTITY)
      def _wpart():
        pltpu.make_async_copy(
            ent_t_hbm.at[:, pl.ds(0, PART_COLS)],
            cb.at[:, pl.ds(0, PART_COLS)], sem).wait()

      col0_rel = j * CHUNK_COLS

      def wlvec(v, carry):
        w16 = wl[pl.ds(v * LANES, LANES)]
        off = lax.shift_right_logical(w16, 16)
        loc = off - col0_rel
        m = ((loc >= 0) & (loc < CHUNK_COLS)
             & (v * LANES + lane < cnt))
        k = jnp.sum(m.astype(jnp.int32))

        def found(carry):
          pend, sv = carry
          pos = plsc.cumsum(m.astype(jnp.int32)) - 1
          plsc.store_scatter(ctmp, [pos], w16, mask=m)
          cvec = ctmp[pl.ds(0, LANES)]

          def one(i, carry):
            pend, sv = carry
            sel = (lane == i).astype(jnp.int32)
            p_i = jnp.sum(sel * cvec)
            loc_i = lax.shift_right_logical(p_i, 16) - col0_rel
            e_i = p_i & 0xFFFF
            slot = lax.rem(pend, LANES)
            cur = lax.rem(lax.shift_right_logical(pend, 4), 2)
            gcol = lo + col0_rel + loc_i

            def extract_into(asmb):
              @pl.when(gcol < TAIL_START)
              def _from_chunk():
                colsplat = jnp.full((LANES,), loc_i, jnp.int32)
                for c4 in range(HIDDEN // LANES):
                  vals = plsc.load_gather(cb, [c4 * LANES + lane, colsplat])
                  asmb[slot, pl.ds(c4 * LANES, LANES)] = vals

              @pl.when(gcol >= TAIL_START)
              def _from_tail():
                tloc = gcol - TAIL_START
                fbase = (lax.shift_right_logical(tloc, 1) * 128
                         + (tloc & 1) * HIDDEN)
                for c4 in range(HIDDEN // LANES):
                  flat = jnp.full((LANES,), fbase + c4 * LANES,
                                  jnp.int32) + lane
                  vals = plsc.load_gather(tailbuf, [flat])
                  asmb[slot, pl.ds(c4 * LANES, LANES)] = vals

            @pl.when(cur == 0)
            def _e0():
              extract_into(asm)

            @pl.when(cur == 1)
            def _e1():
              extract_into(asm1)

            sv = jnp.where(lane == slot, e_i, sv)

            @pl.when(slot == LANES - 1)
            def _flush():
              @pl.when(cur == 0)
              def _f0():
                pltpu.async_copy(asm, stage_hbm.at[sv], sem2)

              @pl.when(cur == 1)
              def _f1():
                pltpu.async_copy(asm1, stage_hbm.at[sv], sem2)

              @pl.when(pend >= 2 * LANES - 1)
              def _drain_prev():
                pltpu.make_async_copy(
                    stage_hbm.at[pl.ds(0, LANES)], asm, sem2).wait()

            sv = jnp.where(slot == LANES - 1, dump_ids, sv)
            return pend + 1, sv

          return lax.fori_loop(0, k, one, (pend, sv))

        return found(carry)

      return lax.fori_loop(0, nvec, wlvec, carry)

    def chunk_pair(jp, carry):
      j0 = jp * 2

      def maybe_fire(j, cb, sem):
        @pl.when(j < nch)
        def _():
          fire(j, cb, sem)

      carry = process(j0, cb0, sem0, carry)
      maybe_fire(j0 + 2, cb0, sem0)
      carry = lax.cond(j0 + 1 < nch,
                       lambda c: process(j0 + 1, cb1, sem1, c),
                       lambda c: c, carry)
      maybe_fire(j0 + 3, cb1, sem1)
      return carry

    fire(0, cb0, sem0)
    fire(1, cb1, sem1)
    nch_pair = (nch + 1) // 2
    pend, sv = lax.fori_loop(0, nch_pair, chunk_pair,
                             (jnp.int32(0), dump_ids))

    @pl.when(pend >= LANES)
    def _drain_last():
      pltpu.make_async_copy(
          stage_hbm.at[pl.ds(0, LANES)], asm, sem2).wait()

    @pl.when(lax.rem(pend, LANES) != 0)
    def _final_flush():
      curf = lax.rem(lax.shift_right_logical(pend, 4), 2)

      @pl.when(curf == 0)
      def _p0():
        pltpu.sync_copy(asm, stage_hbm.at[sv])

      @pl.when(curf == 1)
      def _p1():
        pltpu.sync_copy(asm1, stage_hbm.at[sv])

  return gather_kernel


def _make_score_kernel():
  mesh = plsc.VectorSubcoreMesh(core_axis_name="c", subcore_axis_name="s")
  GRP = 64
  NGRP = B_PER_W // GRP

  @functools.partial(
      pl.kernel,
      mesh=mesh,
      compiler_params=pltpu.CompilerParams(needs_layout_passes=False),
      out_type=jax.ShapeDtypeStruct((NUM_WORKERS, B_PER_W), jnp.float32),
      scratch_types=[
          pltpu.VMEM((B_PER_W,), jnp.int32),
          pltpu.VMEM((GRP // 16, 16), jnp.int32),
          pltpu.VMEM((GRP, 128), jnp.float32),
          pltpu.VMEM((GRP, 128), jnp.float32),
          pltpu.VMEM((GRP, 128), jnp.float32),
          pltpu.VMEM((B_PER_W,), jnp.float32),
          pltpu.SemaphoreType.DMA,
      ],
  )
  def score_kernel(rels_hbm, stage_hbm, relp_hbm, out_hbm,
                   relidx, pidx, hbuf, tbuf, rbuf, out_v, sem):
    wid = lax.axis_index("s") * 2 + lax.axis_index("c")
    base = wid * B_PER_W
    lane = lax.iota(jnp.int32, LANES)

    pltpu.sync_copy(rels_hbm.at[wid], relidx)

    def group(g, _):
      gb = g * GRP
      def mkpidx(q, _):
        pidx[q, :] = lax.shift_right_logical(
            relidx[pl.ds(gb + q * 16, 16)], 1)
        return 0
      lax.fori_loop(0, GRP // 16, mkpidx, 0)

      pltpu.async_copy(stage_hbm.at[pl.ds(base + gb, GRP)], hbuf, sem)
      pltpu.async_copy(stage_hbm.at[pl.ds(BATCH + base + gb, GRP)], tbuf,
                       sem)
      for q in range(GRP // 16):
        pltpu.async_copy(relp_hbm.at[pidx.at[q]],
                         rbuf.at[pl.ds(q * 16, 16)], sem)
      pltpu.make_async_copy(stage_hbm.at[pl.ds(0, GRP)], hbuf, sem).wait()
      pltpu.make_async_copy(stage_hbm.at[pl.ds(0, GRP)], tbuf, sem).wait()
      for q in range(GRP // 16):
        pltpu.make_async_copy(stage_hbm.at[pl.ds(0, 16)],
                              rbuf.at[pl.ds(q * 16, 16)], sem).wait()

      def sample16(q, _):
        rv = relidx[pl.ds(gb + q * LANES, LANES)]
        parity = rv & 1
        scores = jnp.zeros((LANES,), jnp.float32)
        for s in range(LANES):
          i = q * LANES + s
          sel = (lane == s).astype(jnp.int32)
          pbase = jnp.sum(sel * parity) * HIDDEN
          acc = jnp.zeros((LANES,), jnp.float32)
          for c4 in range(HIDDEN // LANES):
            h = hbuf[i, pl.ds(c4 * LANES, LANES)]
            t = tbuf[i, pl.ds(c4 * LANES, LANES)]
            r = rbuf[i, pl.ds(pbase + c4 * LANES, LANES)]
            acc = acc + jnp.abs(h + r - t)
          scores = jnp.where(lane == s, GAMMA - jnp.sum(acc), scores)
        out_v[pl.ds(gb + q * LANES, LANES)] = scores
        return 0

      lax.fori_loop(0, GRP // LANES, sample16, 0)
      return 0

    lax.fori_loop(0, NGRP, group, 0)
    pltpu.sync_copy(out_v, out_hbm.at[wid])

  return score_kernel


_GATHER = _make_gather_kernel()
_SCORE = _make_score_kernel()


@jax.jit
def kernel(sample, entity_embedding, relation_embedding):
  heads = sample[:, 0]
  rels = sample[:, 1].reshape(NUM_WORKERS, B_PER_W)
  tails = sample[:, 2]
  ent_t = entity_embedding.T
  ent_tail = entity_embedding[NENTITY - 64:].reshape(-1)
  relp = relation_embedding.reshape(-1, 2 * HIDDEN)
  stage = _GATHER(heads, tails, ent_t, ent_tail)
  out = _SCORE(rels, stage, relp)
  return out.reshape(BATCH, 1)


# speedup vs baseline: 1.1322x; 1.1322x over previous
import functools

import jax
import jax.numpy as jnp
from jax import lax
from jax.experimental import pallas as pl
from jax.experimental.pallas import tpu as pltpu
from jax.experimental.pallas import tpu_sc as plsc

GAMMA = 12.0
HIDDEN = 64
BATCH = 16384
LANES = 16
NUM_WORKERS = 32
B_PER_W = BATCH // NUM_WORKERS
NENTITY = 1000000
NTILES = (NENTITY + 127) // 128
BIG_W = NTILES - (NTILES // NUM_WORKERS) * NUM_WORKERS
TILES_SMALL = NTILES // NUM_WORKERS
CHUNK_COLS = 512
N_ENTRIES = 2 * BATCH
STAGE_ROWS = N_ENTRIES + LANES
IDX_CHUNK = 4096


def _worker_range(wid):
  tiles = jnp.where(wid < BIG_W, TILES_SMALL + 1, TILES_SMALL)
  start = jnp.where(
      wid < BIG_W,
      wid * (TILES_SMALL + 1),
      BIG_W * (TILES_SMALL + 1) + (wid - BIG_W) * TILES_SMALL)
  return start * 128, tiles * 128


def _make_gather_kernel():
  mesh = plsc.VectorSubcoreMesh(core_axis_name="c", subcore_axis_name="s")

  @functools.partial(
      pl.kernel,
      mesh=mesh,
      compiler_params=pltpu.CompilerParams(needs_layout_passes=False),
      out_type=jax.ShapeDtypeStruct((STAGE_ROWS, 128), jnp.float32),
      scratch_types=[
          pltpu.VMEM((IDX_CHUNK,), jnp.int32),
          pltpu.VMEM((N_ENTRIES + 64,), jnp.int32),
          pltpu.VMEM((HIDDEN, CHUNK_COLS), jnp.float32),
          pltpu.VMEM((HIDDEN, CHUNK_COLS), jnp.float32),
          pltpu.VMEM((LANES, 128), jnp.float32),
          pltpu.VMEM((LANES, 128), jnp.float32),
          pltpu.VMEM((32 * 128,), jnp.float32),
          pltpu.VMEM((4 * LANES,), jnp.int32),
          pltpu.SemaphoreType.DMA,
          pltpu.SemaphoreType.DMA,
          pltpu.SemaphoreType.DMA,
      ],
  )
  def gather_kernel(heads_hbm, tails_hbm, ent_t_hbm, tail_hbm, stage_hbm,
                    idxbuf, wl, cb0, cb1, asm, asm1, tailbuf, ctmp,
                    sem0, sem1, sem2):
    wid = lax.axis_index("s") * 2 + lax.axis_index("c")
    lo, rlen = _worker_range(wid)
    lane = lax.iota(jnp.int32, LANES)
    dump_ids = jnp.full((LANES,), N_ENTRIES, jnp.int32) + lane
    pltpu.sync_copy(tail_hbm, tailbuf)

    def scan_table(tab_hbm, entry_base, cnt0):
      def piece(pc, cntv):
        pltpu.sync_copy(tab_hbm.at[pl.ds(pc * IDX_CHUNK, IDX_CHUNK)], idxbuf)
        def vec(v, cntv):
          idx16 = idxbuf[pl.ds(v * LANES, LANES)]
          off = idx16 - lo
          m = (off >= 0) & (off < rlen)
          packed = lax.shift_left(off, 16) | (
              entry_base + pc * IDX_CHUNK + v * LANES + lane)
          pos = cntv + plsc.cumsum(m.astype(jnp.int32)) - 1
          plsc.store_scatter(wl, [pos], packed, mask=m)
          return cntv + plsc.all_reduce_population_count(m)
        return lax.fori_loop(0, IDX_CHUNK // LANES, vec, cntv)
      return lax.fori_loop(0, BATCH // IDX_CHUNK, piece, cnt0)

    cntv = scan_table(heads_hbm, 0, jnp.zeros((LANES,), jnp.int32))
    cntv = scan_table(tails_hbm, BATCH, cntv)
    cnt = jnp.sum(jnp.where(lane == 0, cntv, 0))
    nvec = (cnt + LANES - 1) // LANES

    nch = (rlen + CHUNK_COLS - 1) // CHUNK_COLS
    PART_COLS = 384
    TAIL_START = NENTITY - 64

    def fire(j, cb, sem):
      col0 = lo + j * CHUNK_COLS

      @pl.when(col0 + CHUNK_COLS <= NENTITY)
      def _full():
        pltpu.async_copy(ent_t_hbm.at[:, pl.ds(col0, CHUNK_COLS)], cb, sem)

      @pl.when(col0 + CHUNK_COLS > NENTITY)
      def _part():
        pltpu.async_copy(ent_t_hbm.at[:, pl.ds(col0, PART_COLS)],
                         cb.at[:, pl.ds(0, PART_COLS)], sem)

    def process(j, cb, sem, carry):
      col0 = lo + j * CHUNK_COLS

      @pl.when(col0 + CHUNK_COLS <= NENTITY)
      def _wfull():
        pltpu.make_async_copy(
            ent_t_hbm.at[:, pl.ds(0, CHUNK_COLS)], cb, sem).wait()

      @pl.when(col0 + CHUNK_COLS > NENTITY)
      def _wpart():
        pltpu.make_async_copy(
            ent_t_hbm.at[:, pl.ds(0, PART_COLS)],
            cb.at[:, pl.ds(0, PART_COLS)], sem).wait()

      col0_rel = j * CHUNK_COLS

      def wlvec(v4, carry):
        ks = []
        for u in range(4):
          v = v4 * 4 + u
          w16 = wl[pl.ds(v * LANES, LANES)]
          off = lax.shift_right_logical(w16, 16)
          loc = off - col0_rel
          m = ((loc >= 0) & (loc < CHUNK_COLS)
               & (v * LANES + lane < cnt))
          pos = plsc.cumsum(m.astype(jnp.int32)) - 1 + u * LANES
          plsc.store_scatter(ctmp, [pos], w16, mask=m)
          ks.append(jnp.sum(m.astype(jnp.int32)))

        def found(u, k, carry):
          cvec = ctmp[pl.ds(u * LANES, LANES)]

          def one(i, carry):
            pend, sv = carry
            sel = (lane == i).astype(jnp.int32)
            p_i = jnp.sum(sel * cvec)
            loc_i = lax.shift_right_logical(p_i, 16) - col0_rel
            e_i = p_i & 0xFFFF
            slot = lax.rem(pend, LANES)
            cur = lax.rem(lax.shift_right_logical(pend, 4), 2)
            gcol = lo + col0_rel + loc_i

            def extract_into(asmb):
              @pl.when(gcol < TAIL_START)
              def _from_chunk():
                colsplat = jnp.full((LANES,), loc_i, jnp.int32)
                for c4 in range(HIDDEN // LANES):
                  vals = plsc.load_gather(cb, [c4 * LANES + lane, colsplat])
                  asmb[slot, pl.ds(c4 * LANES, LANES)] = vals

              @pl.when(gcol >= TAIL_START)
              def _from_tail():
                tloc = gcol - TAIL_START
                fbase = (lax.shift_right_logical(tloc, 1) * 128
                         + (tloc & 1) * HIDDEN)
                for c4 in range(HIDDEN // LANES):
                  flat = jnp.full((LANES,), fbase + c4 * LANES,
                                  jnp.int32) + lane
                  vals = plsc.load_gather(tailbuf, [flat])
                  asmb[slot, pl.ds(c4 * LANES, LANES)] = vals

            @pl.when(cur == 0)
            def _e0():
              extract_into(asm)

            @pl.when(cur == 1)
            def _e1():
              extract_into(asm1)

            sv = jnp.where(lane == slot, e_i, sv)

            @pl.when(slot == LANES - 1)
            def _flush():
              @pl.when(cur == 0)
              def _f0():
                pltpu.async_copy(asm, stage_hbm.at[sv], sem2)

              @pl.when(cur == 1)
              def _f1():
                pltpu.async_copy(asm1, stage_hbm.at[sv], sem2)

              @pl.when(pend >= 2 * LANES - 1)
              def _drain_prev():
                pltpu.make_async_copy(
                    stage_hbm.at[pl.ds(0, LANES)], asm, sem2).wait()

            sv = jnp.where(slot == LANES - 1, dump_ids, sv)
            return pend + 1, sv

          return lax.fori_loop(0, k, one, carry)

        for u in range(4):
          carry = lax.cond(ks[u] > 0,
                           functools.partial(found, u, ks[u]),
                           lambda c: c, carry)
        return carry

      return lax.fori_loop(0, (nvec + 3) // 4, wlvec, carry)

    def chunk_pair(jp, carry):
      j0 = jp * 2

      def maybe_fire(j, cb, sem):
        @pl.when(j < nch)
        def _():
          fire(j, cb, sem)

      carry = process(j0, cb0, sem0, carry)
      maybe_fire(j0 + 2, cb0, sem0)
      carry = lax.cond(j0 + 1 < nch,
                       lambda c: process(j0 + 1, cb1, sem1, c),
                       lambda c: c, carry)
      maybe_fire(j0 + 3, cb1, sem1)
      return carry

    fire(0, cb0, sem0)
    fire(1, cb1, sem1)
    nch_pair = (nch + 1) // 2
    pend, sv = lax.fori_loop(0, nch_pair, chunk_pair,
                             (jnp.int32(0), dump_ids))

    @pl.when(pend >= LANES)
    def _drain_last():
      pltpu.make_async_copy(
          stage_hbm.at[pl.ds(0, LANES)], asm, sem2).wait()

    @pl.when(lax.rem(pend, LANES) != 0)
    def _final_flush():
      curf = lax.rem(lax.shift_right_logical(pend, 4), 2)

      @pl.when(curf == 0)
      def _p0():
        pltpu.sync_copy(asm, stage_hbm.at[sv])

      @pl.when(curf == 1)
      def _p1():
        pltpu.sync_copy(asm1, stage_hbm.at[sv])

  return gather_kernel


def _make_score_kernel():
  mesh = plsc.VectorSubcoreMesh(core_axis_name="c", subcore_axis_name="s")
  GRP = 64
  NGRP = B_PER_W // GRP

  @functools.partial(
      pl.kernel,
      mesh=mesh,
      compiler_params=pltpu.CompilerParams(needs_layout_passes=False),
      out_type=jax.ShapeDtypeStruct((NUM_WORKERS, B_PER_W), jnp.float32),
      scratch_types=[
          pltpu.VMEM((B_PER_W,), jnp.int32),
          pltpu.VMEM((GRP // 16, 16), jnp.int32),
          pltpu.VMEM((GRP, 128), jnp.float32),
          pltpu.VMEM((GRP, 128), jnp.float32),
          pltpu.VMEM((GRP, 128), jnp.float32),
          pltpu.VMEM((B_PER_W,), jnp.float32),
          pltpu.SemaphoreType.DMA,
      ],
  )
  def score_kernel(rels_hbm, stage_hbm, relp_hbm, out_hbm,
                   relidx, pidx, hbuf, tbuf, rbuf, out_v, sem):
    wid = lax.axis_index("s") * 2 + lax.axis_index("c")
    base = wid * B_PER_W
    lane = lax.iota(jnp.int32, LANES)

    pltpu.sync_copy(rels_hbm.at[wid], relidx)

    def group(g, _):
      gb = g * GRP
      def mkpidx(q, _):
        pidx[q, :] = lax.shift_right_logical(
            relidx[pl.ds(gb + q * 16, 16)], 1)
        return 0
      lax.fori_loop(0, GRP // 16, mkpidx, 0)

      pltpu.async_copy(stage_hbm.at[pl.ds(base + gb, GRP)], hbuf, sem)
      pltpu.async_copy(stage_hbm.at[pl.ds(BATCH + base + gb, GRP)], tbuf,
                       sem)
      for q in range(GRP // 16):
        pltpu.async_copy(relp_hbm.at[pidx.at[q]],
                         rbuf.at[pl.ds(q * 16, 16)], sem)
      pltpu.make_async_copy(stage_hbm.at[pl.ds(0, GRP)], hbuf, sem).wait()
      pltpu.make_async_copy(stage_hbm.at[pl.ds(0, GRP)], tbuf, sem).wait()
      for q in range(GRP // 16):
        pltpu.make_async_copy(stage_hbm.at[pl.ds(0, 16)],
                              rbuf.at[pl.ds(q * 16, 16)], sem).wait()

      def sample16(q, _):
        rv = relidx[pl.ds(gb + q * LANES, LANES)]
        parity = rv & 1
        scores = jnp.zeros((LANES,), jnp.float32)
        for s in range(LANES):
          i = q * LANES + s
          sel = (lane == s).astype(jnp.int32)
          pbase = jnp.sum(sel * parity) * HIDDEN
          acc = jnp.zeros((LANES,), jnp.float32)
          for c4 in range(HIDDEN // LANES):
            h = hbuf[i, pl.ds(c4 * LANES, LANES)]
            t = tbuf[i, pl.ds(c4 * LANES, LANES)]
            r = rbuf[i, pl.ds(pbase + c4 * LANES, LANES)]
            acc = acc + jnp.abs(h + r - t)
          scores = jnp.where(lane == s, GAMMA - jnp.sum(acc), scores)
        out_v[pl.ds(gb + q * LANES, LANES)] = scores
        return 0

      lax.fori_loop(0, GRP // LANES, sample16, 0)
      return 0

    lax.fori_loop(0, NGRP, group, 0)
    pltpu.sync_copy(out_v, out_hbm.at[wid])

  return score_kernel


_GATHER = _make_gather_kernel()
_SCORE = _make_score_kernel()


@jax.jit
def kernel(sample, entity_embedding, relation_embedding):
  heads = sample[:, 0]
  rels = sample[:, 1].reshape(NUM_WORKERS, B_PER_W)
  tails = sample[:, 2]
  ent_t = entity_embedding.T
  ent_tail = entity_embedding[NENTITY - 64:].reshape(-1)
  relp = relation_embedding.reshape(-1, 2 * HIDDEN)
  stage = _GATHER(heads, tails, ent_t, ent_tail)
  out = _SCORE(rels, stage, relp)
  return out.reshape(BATCH, 1)
